# baseline (speedup 1.0000x reference)
import functools

import jax
import jax.numpy as jnp
from jax import lax
from jax.experimental import pallas as pl
from jax.experimental.pallas import tpu as pltpu
from jax.experimental.pallas import tpu_sc as plsc

T = 8192
D = 2048
E = 64
NC = 2
NS = 16
L = 16
NW = NC * NS
CHUNKS = 2
TPC = T // CHUNKS
RPW = TPC // NW
G = RPW // L
TB = 256
BLK = TB // RPW


def _logits_body(w_ref, x_ref, out_ref):
    res = lax.dot_general(
        w_ref[...], x_ref[...],
        dimension_numbers=(((1,), (1,)), ((), ())),
        preferred_element_type=jnp.float32,
    )
    for b in range(BLK):
        out_ref[b] = res[:, b * RPW:(b + 1) * RPW]


def _compute_logits(x, W, chunk):
    nsteps = TPC // TB
    base = chunk * nsteps
    return pl.pallas_call(
        _logits_body,
        grid=(nsteps,),
        in_specs=[
            pl.BlockSpec((E, D), lambda i: (0, 0)),
            pl.BlockSpec((TB, D), lambda i, _b=base: (_b + i, 0)),
        ],
        out_specs=pl.BlockSpec((BLK, E, RPW), lambda i: (i, 0, 0)),
        out_shape=jax.ShapeDtypeStruct((NW, E, RPW), jnp.float32),
    )(W, x)


_mesh = plsc.VectorSubcoreMesh(core_axis_name="c", subcore_axis_name="s")


@functools.partial(
    pl.kernel,
    mesh=_mesh,
    out_type=[
        jax.ShapeDtypeStruct((2, TPC), jnp.int32),
        jax.ShapeDtypeStruct((2, TPC), jnp.float32),
    ],
    scratch_types=[
        pltpu.VMEM((E, RPW), jnp.float32),
        pltpu.VMEM((RPW,), jnp.int32),
        pltpu.VMEM((RPW,), jnp.int32),
        pltpu.VMEM((RPW,), jnp.float32),
        pltpu.VMEM((RPW,), jnp.float32),
    ],
)
def _route(logits_hbm, idx_hbm, w_hbm, buf, i1_v, i2_v, w1_v, w2_v):
    wid = lax.axis_index("s") * NC + lax.axis_index("c")
    pltpu.sync_copy(logits_hbm.at[wid], buf)
    for g in range(G):
        neg = jnp.full((L,), -3.0e38, jnp.float32)
        zero = jnp.zeros((L,), jnp.int32)

        def step(e, carry, _g=g):
            v1, i1, v2, i2 = carry
            c = buf[e, pl.ds(_g * L, L)]
            ei = jnp.full((L,), 0, jnp.int32) + e
            t2 = c > v2
            mv2 = jnp.where(t2, c, v2)
            mi2 = jnp.where(t2, ei, i2)
            t1 = c > v1
            nv2 = jnp.where(t1, v1, mv2)
            ni2 = jnp.where(t1, i1, mi2)
            nv1 = jnp.where(t1, c, v1)
            ni1 = jnp.where(t1, ei, i1)
            return nv1, ni1, nv2, ni2

        v1, i1, v2, i2 = lax.fori_loop(0, E, step, (neg, zero, neg, zero))
        ed = jnp.exp(v2 - v1)
        denom = 1.0 + ed
        sl = pl.ds(g * L, L)
        i1_v[sl] = i1
        i2_v[sl] = i2
        w1_v[sl] = 1.0 / denom
        w2_v[sl] = ed / denom
    base = wid * RPW
    pltpu.sync_copy(i1_v, idx_hbm.at[0, pl.ds(base, RPW)])
    pltpu.sync_copy(i2_v, idx_hbm.at[1, pl.ds(base, RPW)])
    pltpu.sync_copy(w1_v, w_hbm.at[0, pl.ds(base, RPW)])
    pltpu.sync_copy(w2_v, w_hbm.at[1, pl.ds(base, RPW)])


def kernel(x, W, top_k):
    idx_parts, w_parts = [], []
    for c in range(CHUNKS):
        logits = _compute_logits(x, W, c)
        idx_pl, w_pl = _route(logits)
        idx_parts.append(idx_pl)
        w_parts.append(w_pl)
    idx_pl = jnp.concatenate(idx_parts, axis=1)
    w_pl = jnp.concatenate(w_parts, axis=1)
    topk_idx = jnp.stack([idx_pl[0], idx_pl[1]], axis=-1).astype(jnp.int64)
    topk_w = jnp.stack([w_pl[0], w_pl[1]], axis=-1)
    return topk_idx, topk_w


# speedup vs baseline: 1.1586x; 1.1586x over previous
import functools

import jax
import jax.numpy as jnp
from jax import lax
from jax.experimental import pallas as pl
from jax.experimental.pallas import tpu as pltpu
from jax.experimental.pallas import tpu_sc as plsc

T = 8192
D = 2048
E = 64
NC = 2
NS = 16
L = 16
NW = NC * NS
CHUNKS = 2
TPC = T // CHUNKS
RPW = TPC // NW
G = RPW // L
TB = 512
BLK = TB // RPW


def _logits_body(w_ref, x_ref, out_ref):
    res = lax.dot_general(
        w_ref[...], x_ref[...],
        dimension_numbers=(((1,), (1,)), ((), ())),
        preferred_element_type=jnp.float32,
    )
    for b in range(BLK):
        out_ref[b] = res[:, b * RPW:(b + 1) * RPW]


def _compute_logits(x, W, chunk):
    nsteps = TPC // TB
    base = chunk * nsteps
    return pl.pallas_call(
        _logits_body,
        grid=(nsteps,),
        in_specs=[
            pl.BlockSpec((E, D), lambda i: (0, 0)),
            pl.BlockSpec((TB, D), lambda i, _b=base: (_b + i, 0)),
        ],
        out_specs=pl.BlockSpec((BLK, E, RPW), lambda i: (i, 0, 0)),
        out_shape=jax.ShapeDtypeStruct((NW, E, RPW), jnp.float32),
    )(W, x)


_mesh = plsc.VectorSubcoreMesh(core_axis_name="c", subcore_axis_name="s")


@functools.partial(
    pl.kernel,
    mesh=_mesh,
    out_type=[
        jax.ShapeDtypeStruct((2, TPC), jnp.int32),
        jax.ShapeDtypeStruct((2, TPC), jnp.float32),
    ],
    scratch_types=[
        pltpu.VMEM((E, RPW), jnp.float32),
        pltpu.VMEM((RPW,), jnp.int32),
        pltpu.VMEM((RPW,), jnp.int32),
        pltpu.VMEM((RPW,), jnp.float32),
        pltpu.VMEM((RPW,), jnp.float32),
    ],
)
def _route(logits_hbm, idx_hbm, w_hbm, buf, i1_v, i2_v, w1_v, w2_v):
    wid = lax.axis_index("s") * NC + lax.axis_index("c")
    pltpu.sync_copy(logits_hbm.at[wid], buf)
    for g in range(G):
        neg = jnp.full((L,), -3.0e38, jnp.float32)
        zero = jnp.zeros((L,), jnp.int32)

        def step(e, carry, _g=g):
            v1, i1, v2, i2 = carry
            c = buf[e, pl.ds(_g * L, L)]
            ei = jnp.full((L,), 0, jnp.int32) + e
            t2 = c > v2
            mv2 = jnp.where(t2, c, v2)
            mi2 = jnp.where(t2, ei, i2)
            t1 = c > v1
            nv2 = jnp.where(t1, v1, mv2)
            ni2 = jnp.where(t1, i1, mi2)
            nv1 = jnp.where(t1, c, v1)
            ni1 = jnp.where(t1, ei, i1)
            return nv1, ni1, nv2, ni2

        v1, i1, v2, i2 = lax.fori_loop(0, E, step, (neg, zero, neg, zero))
        ed = jnp.exp(v2 - v1)
        denom = 1.0 + ed
        sl = pl.ds(g * L, L)
        i1_v[sl] = i1
        i2_v[sl] = i2
        w1_v[sl] = 1.0 / denom
        w2_v[sl] = ed / denom
    base = wid * RPW
    pltpu.sync_copy(i1_v, idx_hbm.at[0, pl.ds(base, RPW)])
    pltpu.sync_copy(i2_v, idx_hbm.at[1, pl.ds(base, RPW)])
    pltpu.sync_copy(w1_v, w_hbm.at[0, pl.ds(base, RPW)])
    pltpu.sync_copy(w2_v, w_hbm.at[1, pl.ds(base, RPW)])


def kernel(x, W, top_k):
    idx_parts, w_parts = [], []
    for c in range(CHUNKS):
        logits = _compute_logits(x, W, c)
        idx_pl, w_pl = _route(logits)
        idx_parts.append(idx_pl)
        w_parts.append(w_pl)
    idx_pl = jnp.concatenate(idx_parts, axis=1)
    w_pl = jnp.concatenate(w_parts, axis=1)
    topk_idx = jnp.stack([idx_pl[0], idx_pl[1]], axis=-1).astype(jnp.int64)
    topk_w = jnp.stack([w_pl[0], w_pl[1]], axis=-1)
    return topk_idx, topk_w


# speedup vs baseline: 1.2079x; 1.0425x over previous
import functools

import jax
import jax.numpy as jnp
from jax import lax
from jax.experimental import pallas as pl
from jax.experimental.pallas import tpu as pltpu
from jax.experimental.pallas import tpu_sc as plsc

T = 8192
D = 2048
E = 64
NC = 2
NS = 16
L = 16
NW = NC * NS
CHUNKS = 2
TPC = T // CHUNKS
RPW = TPC // NW
G = RPW // L
TB = 1024
BLK = TB // RPW


def _logits_body(w_ref, x_ref, out_ref):
    res = lax.dot_general(
        w_ref[...], x_ref[...],
        dimension_numbers=(((1,), (1,)), ((), ())),
        preferred_element_type=jnp.float32,
    )
    for b in range(BLK):
        out_ref[b] = res[:, b * RPW:(b + 1) * RPW]


def _compute_logits(x, W, chunk):
    nsteps = TPC // TB
    base = chunk * nsteps
    return pl.pallas_call(
        _logits_body,
        grid=(nsteps,),
        in_specs=[
            pl.BlockSpec((E, D), lambda i: (0, 0)),
            pl.BlockSpec((TB, D), lambda i, _b=base: (_b + i, 0)),
        ],
        out_specs=pl.BlockSpec((BLK, E, RPW), lambda i: (i, 0, 0)),
        out_shape=jax.ShapeDtypeStruct((NW, E, RPW), jnp.float32),
    )(W, x)


_mesh = plsc.VectorSubcoreMesh(core_axis_name="c", subcore_axis_name="s")


@functools.partial(
    pl.kernel,
    mesh=_mesh,
    out_type=[
        jax.ShapeDtypeStruct((2, TPC), jnp.int32),
        jax.ShapeDtypeStruct((2, TPC), jnp.float32),
    ],
    scratch_types=[
        pltpu.VMEM((E, RPW), jnp.float32),
        pltpu.VMEM((RPW,), jnp.int32),
        pltpu.VMEM((RPW,), jnp.int32),
        pltpu.VMEM((RPW,), jnp.float32),
        pltpu.VMEM((RPW,), jnp.float32),
    ],
)
def _route(logits_hbm, idx_hbm, w_hbm, buf, i1_v, i2_v, w1_v, w2_v):
    wid = lax.axis_index("s") * NC + lax.axis_index("c")
    pltpu.sync_copy(logits_hbm.at[wid], buf)
    for g in range(G):
        neg = jnp.full((L,), -3.0e38, jnp.float32)
        zero = jnp.zeros((L,), jnp.int32)

        def step(e, carry, _g=g):
            v1, i1, v2, i2 = carry
            c = buf[e, pl.ds(_g * L, L)]
            ei = jnp.full((L,), 0, jnp.int32) + e
            t2 = c > v2
            mv2 = jnp.where(t2, c, v2)
            mi2 = jnp.where(t2, ei, i2)
            t1 = c > v1
            nv2 = jnp.where(t1, v1, mv2)
            ni2 = jnp.where(t1, i1, mi2)
            nv1 = jnp.where(t1, c, v1)
            ni1 = jnp.where(t1, ei, i1)
            return nv1, ni1, nv2, ni2

        v1, i1, v2, i2 = lax.fori_loop(0, E, step, (neg, zero, neg, zero))
        ed = jnp.exp(v2 - v1)
        denom = 1.0 + ed
        sl = pl.ds(g * L, L)
        i1_v[sl] = i1
        i2_v[sl] = i2
        w1_v[sl] = 1.0 / denom
        w2_v[sl] = ed / denom
    base = wid * RPW
    pltpu.sync_copy(i1_v, idx_hbm.at[0, pl.ds(base, RPW)])
    pltpu.sync_copy(i2_v, idx_hbm.at[1, pl.ds(base, RPW)])
    pltpu.sync_copy(w1_v, w_hbm.at[0, pl.ds(base, RPW)])
    pltpu.sync_copy(w2_v, w_hbm.at[1, pl.ds(base, RPW)])


def kernel(x, W, top_k):
    idx_parts, w_parts = [], []
    for c in range(CHUNKS):
        logits = _compute_logits(x, W, c)
        idx_pl, w_pl = _route(logits)
        idx_parts.append(idx_pl)
        w_parts.append(w_pl)
    idx_pl = jnp.concatenate(idx_parts, axis=1)
    w_pl = jnp.concatenate(w_parts, axis=1)
    topk_idx = jnp.stack([idx_pl[0], idx_pl[1]], axis=-1).astype(jnp.int64)
    topk_w = jnp.stack([w_pl[0], w_pl[1]], axis=-1)
    return topk_idx, topk_w
